# baseline (speedup 1.0000x reference)
import functools

import jax
import jax.numpy as jnp
from jax import lax
from jax.experimental import pallas as pl
from jax.experimental.pallas import tpu as pltpu
from jax.experimental.pallas import tpu_sc as plsc

N_NODES = 100000
N_EDGES = 6400000
EPS = 1e-15

NC = 2
NS = 16
NW = NC * NS

NPAD = 100096
RPT = NPAD // NS
B = 5000
EPW = N_EDGES // NW
NCHUNK = EPW // B
NITER = NCHUNK // 4

_mesh = plsc.VectorSubcoreMesh(core_axis_name="c", subcore_axis_name="s")


@functools.partial(
    pl.kernel,
    out_type=(jax.ShapeDtypeStruct((NC, NPAD), jnp.float32),
              jax.ShapeDtypeStruct((NC, NPAD), jnp.float32)),
    mesh=_mesh,
    compiler_params=pltpu.CompilerParams(use_tc_tiling_on_sc=False),
    scratch_types=[
        pltpu.VMEM_SHARED((NPAD,), jnp.float32),
        pltpu.VMEM_SHARED((NPAD,), jnp.float32),
        pltpu.VMEM_SHARED((NPAD,), jnp.float32),
        pltpu.VMEM_SHARED((NPAD,), jnp.float32),
        pltpu.VMEM((4, B), jnp.int32),
        pltpu.VMEM((4, B), jnp.int32),
        pltpu.VMEM((2, B), jnp.float32),
        pltpu.VMEM((2, B), jnp.float32),
        pltpu.SemaphoreType.DMA,
        pltpu.SemaphoreType.DMA,
        pltpu.SemaphoreType.DMA,
        pltpu.SemaphoreType.DMA,
        pltpu.SemaphoreType.DMA,
        pltpu.SemaphoreType.DMA,
        pltpu.SemaphoreType.DMA,
        pltpu.SemaphoreType.DMA,
    ],
)
def _propagate(x0_hbm, x1_hbm, zeros_hbm, src_hbm, dst_hbm, o0_hbm, o1_hbm,
               x0s, x1s, a0s, a1s, sidx, didx, v0, v1,
               si0, si1, si2, si3, sg0, sg1, ss0, ss1):
    c = lax.axis_index("c")
    s = lax.axis_index("s")
    wid = s * NC + c
    ebase = wid * EPW

    sis = (si0, si1, si2, si3)
    sgs = (sg0, sg1)
    sss = (ss0, ss1)

    def _prefetch(i, d):
        pltpu.async_copy(src_hbm.at[pl.ds(ebase + i * B, B)], sidx.at[d], sis[d])
        pltpu.async_copy(dst_hbm.at[pl.ds(ebase + i * B, B)], didx.at[d], sis[d])

    _prefetch(0, 0)
    _prefetch(1, 1)

    base = s * RPT
    sl = pl.ds(base, RPT)
    st = pltpu.async_copy(x0_hbm.at[sl], x0s.at[sl], sg0)
    pltpu.async_copy(x1_hbm.at[sl], x1s.at[sl], sg0)
    pltpu.async_copy(zeros_hbm.at[sl], a0s.at[sl], sg0)
    pltpu.async_copy(zeros_hbm.at[sl], a1s.at[sl], sg0)
    for _ in range(4):
        st.wait()
    plsc.subcore_barrier()

    def _wait2(sem):
        pltpu.make_async_copy(x0_hbm.at[pl.ds(0, B)], v0.at[0], sem).wait()
        pltpu.make_async_copy(x0_hbm.at[pl.ds(0, B)], v0.at[0], sem).wait()

    def _window(i, k, it):
        b = k % 2
        sx, dx = sidx.at[k], didx.at[k]
        g0, g1 = v0.at[b], v1.at[b]
        _wait2(sis[k])
        if k < 2:
            @pl.when(it > 0)
            def _():
                _wait2(sss[b])
        else:
            _wait2(sss[b])
        pltpu.async_copy(x0s.at[sx], g0, sgs[b])
        pltpu.async_copy(x1s.at[sx], g1, sgs[b])
        _wait2(sgs[b])
        pltpu.async_copy(g0, a0s.at[dx], sss[b], add=True)
        pltpu.async_copy(g1, a1s.at[dx], sss[b], add=True)

        @pl.when(i + 2 < NCHUNK)
        def _():
            _prefetch(i + 2, (k + 2) % 4)

    def body(it, carry):
        for k in range(4):
            _window(it * 4 + k, k, it)
        return carry

    lax.fori_loop(0, NITER, body, 0)
    _wait2(ss0)
    _wait2(ss1)
    plsc.subcore_barrier()

    wb = pltpu.async_copy(a0s.at[sl], o0_hbm.at[c].at[sl], sg0)
    pltpu.async_copy(a1s.at[sl], o1_hbm.at[c].at[sl], sg0)
    wb.wait()
    wb.wait()


def _norm_body(x_ref, o_ref):
    xt = jnp.pad(x_ref[...], ((0, NPAD - N_NODES), (0, 0))).T
    x0 = xt[0:1, :]
    x1 = xt[1:2, :]
    nrm = jnp.sqrt(x0 * x0 + x1 * x1)
    o_ref[...] = xt / (nrm + EPS)


def _normalize_tc(x):
    return pl.pallas_call(
        _norm_body,
        out_shape=jax.ShapeDtypeStruct((2, NPAD), jnp.float32),
    )(x)


def _bf16(v):
    return v.astype(jnp.bfloat16).astype(jnp.float32)


def _mid_body(p0_ref, p1_ref, w_ref, o_ref):
    a0 = _bf16(p0_ref[0:1, :] + p0_ref[1:2, :])
    a1 = _bf16(p1_ref[0:1, :] + p1_ref[1:2, :])
    h0 = jnp.maximum(a0 * w_ref[0, 0] + a1 * w_ref[1, 0], 0.0)
    h1 = jnp.maximum(a0 * w_ref[0, 1] + a1 * w_ref[1, 1], 0.0)
    nrm = jnp.sqrt(h0 * h0 + h1 * h1) + EPS
    o_ref[0:1, :] = h0 / nrm
    o_ref[1:2, :] = h1 / nrm


def _mid_tc(p0, p1, w):
    return pl.pallas_call(
        _mid_body,
        out_shape=jax.ShapeDtypeStruct((2, NPAD), jnp.float32),
        in_specs=[
            pl.BlockSpec((NC, NPAD), lambda: (0, 0)),
            pl.BlockSpec((NC, NPAD), lambda: (0, 0)),
            pl.BlockSpec(memory_space=pltpu.SMEM),
        ],
        out_specs=pl.BlockSpec((2, NPAD), lambda: (0, 0)),
    )(p0, p1, w)


def _post_body(p0_ref, p1_ref, w_ref, fw_ref, o_ref):
    a0 = _bf16(p0_ref[0:1, :] + p0_ref[1:2, :])
    a1 = _bf16(p1_ref[0:1, :] + p1_ref[1:2, :])
    h0 = jnp.maximum(a0 * w_ref[0, 0] + a1 * w_ref[1, 0], 0.0)
    h1 = jnp.maximum(a0 * w_ref[0, 1] + a1 * w_ref[1, 1], 0.0)
    o_ref[...] = jax.nn.sigmoid(_bf16(h0) * fw_ref[0] + _bf16(h1) * fw_ref[1])


def _post_tc(p0, p1, w, fw):
    return pl.pallas_call(
        _post_body,
        out_shape=jax.ShapeDtypeStruct((1, NPAD), jnp.float32),
        in_specs=[
            pl.BlockSpec((NC, NPAD), lambda: (0, 0)),
            pl.BlockSpec((NC, NPAD), lambda: (0, 0)),
            pl.BlockSpec(memory_space=pltpu.SMEM),
            pl.BlockSpec(memory_space=pltpu.SMEM),
        ],
        out_specs=pl.BlockSpec((1, NPAD), lambda: (0, 0)),
    )(p0, p1, w, fw)


def kernel(x, edge_index, W1, W2, final_weight):
    src = edge_index[0].astype(jnp.int32)
    dst = edge_index[1].astype(jnp.int32)
    zeros = jnp.zeros((NPAD,), jnp.float32)

    xn = _normalize_tc(x)
    p0, p1 = _propagate(xn[0], xn[1], zeros, src, dst)
    xn2 = _mid_tc(p0, p1, W1)
    q0, q1 = _propagate(xn2[0], xn2[1], zeros, src, dst)
    out = _post_tc(q0, q1, W2, final_weight)
    return out[0, :N_NODES]


# speedup vs baseline: 1.0804x; 1.0804x over previous
import functools

import jax
import jax.numpy as jnp
from jax import lax
from jax.experimental import pallas as pl
from jax.experimental.pallas import tpu as pltpu
from jax.experimental.pallas import tpu_sc as plsc

N_NODES = 100000
N_EDGES = 6400000
EPS = 1e-15

NC = 2
NS = 16
NW = NC * NS

NPAD = 100096
RPT = NPAD // NS
B = 5000
EPW = N_EDGES // NW
NCHUNK = EPW // B
NITER = NCHUNK // 4

_mesh = plsc.VectorSubcoreMesh(core_axis_name="c", subcore_axis_name="s")


@functools.partial(
    pl.kernel,
    out_type=(jax.ShapeDtypeStruct((NC, NPAD), jnp.float32),
              jax.ShapeDtypeStruct((NC, NPAD), jnp.float32)),
    mesh=_mesh,
    compiler_params=pltpu.CompilerParams(use_tc_tiling_on_sc=False),
    scratch_types=[
        pltpu.VMEM_SHARED((NPAD,), jnp.float32),
        pltpu.VMEM_SHARED((NPAD,), jnp.float32),
        pltpu.VMEM_SHARED((NPAD,), jnp.float32),
        pltpu.VMEM_SHARED((NPAD,), jnp.float32),
        pltpu.VMEM((4, B), jnp.int32),
        pltpu.VMEM((4, B), jnp.int32),
        pltpu.VMEM((2, B), jnp.float32),
        pltpu.VMEM((2, B), jnp.float32),
        pltpu.SemaphoreType.DMA,
        pltpu.SemaphoreType.DMA,
        pltpu.SemaphoreType.DMA,
        pltpu.SemaphoreType.DMA,
        pltpu.SemaphoreType.DMA,
        pltpu.SemaphoreType.DMA,
        pltpu.SemaphoreType.DMA,
        pltpu.SemaphoreType.DMA,
    ],
)
def _propagate(x0_hbm, x1_hbm, zeros_hbm, src_hbm, dst_hbm, o0_hbm, o1_hbm,
               x0s, x1s, a0s, a1s, sidx, didx, v0, v1,
               si0, si1, si2, si3, sg0, sg1, ss0, ss1):
    c = lax.axis_index("c")
    s = lax.axis_index("s")
    wid = s * NC + c
    ebase = wid * EPW

    sis = (si0, si1, si2, si3)
    sgs = (sg0, sg1)
    sss = (ss0, ss1)

    def _prefetch(i, d):
        pltpu.async_copy(src_hbm.at[pl.ds(ebase + i * B, B)], sidx.at[d], sis[d])
        pltpu.async_copy(dst_hbm.at[pl.ds(ebase + i * B, B)], didx.at[d], sis[d])

    _prefetch(0, 0)
    _prefetch(1, 1)

    base = s * RPT
    sl = pl.ds(base, RPT)
    st = pltpu.async_copy(x0_hbm.at[sl], x0s.at[sl], sg0)
    pltpu.async_copy(x1_hbm.at[sl], x1s.at[sl], sg0)
    pltpu.async_copy(zeros_hbm.at[sl], a0s.at[sl], sg0)
    pltpu.async_copy(zeros_hbm.at[sl], a1s.at[sl], sg0)
    for _ in range(4):
        st.wait()
    plsc.subcore_barrier()

    def _wait2(sem):
        pltpu.make_async_copy(x0_hbm.at[pl.ds(0, B)], v0.at[0], sem).wait()
        pltpu.make_async_copy(x0_hbm.at[pl.ds(0, B)], v0.at[0], sem).wait()

    def _window(i, k, it):
        b = k % 2
        sx, dx = sidx.at[k], didx.at[k]
        g0, g1 = v0.at[b], v1.at[b]
        _wait2(sis[k])
        if k < 2:
            @pl.when(it > 0)
            def _():
                _wait2(sss[b])
        else:
            _wait2(sss[b])
        pltpu.async_copy(x0s.at[sx], g0, sgs[b])
        pltpu.async_copy(x1s.at[sx], g1, sgs[b])
        _wait2(sgs[b])
        pltpu.async_copy(g0, a0s.at[dx], sss[b], add=True)
        pltpu.async_copy(g1, a1s.at[dx], sss[b], add=True)

        @pl.when(i + 2 < NCHUNK)
        def _():
            _prefetch(i + 2, (k + 2) % 4)

    def body(it, carry):
        for k in range(4):
            _window(it * 4 + k, k, it)
        return carry

    lax.fori_loop(0, NITER, body, 0)
    _wait2(ss0)
    _wait2(ss1)
    plsc.subcore_barrier()

    wb = pltpu.async_copy(a0s.at[sl], o0_hbm.at[c].at[sl], sg0)
    pltpu.async_copy(a1s.at[sl], o1_hbm.at[c].at[sl], sg0)
    wb.wait()
    wb.wait()


def _norm_body(x_ref, o_ref):
    x0 = x_ref[0:1, :]
    x1 = x_ref[1:2, :]
    nrm = jnp.sqrt(x0 * x0 + x1 * x1)
    o_ref[...] = x_ref[...] / (nrm + EPS)


def _normalize_tc(xt):
    return pl.pallas_call(
        _norm_body,
        out_shape=jax.ShapeDtypeStruct((2, NPAD), jnp.float32),
    )(xt)


def _bf16(v):
    return v.astype(jnp.bfloat16).astype(jnp.float32)


def _mid_body(p0_ref, p1_ref, w_ref, o_ref):
    a0 = _bf16(p0_ref[0:1, :] + p0_ref[1:2, :])
    a1 = _bf16(p1_ref[0:1, :] + p1_ref[1:2, :])
    h0 = jnp.maximum(a0 * w_ref[0, 0] + a1 * w_ref[1, 0], 0.0)
    h1 = jnp.maximum(a0 * w_ref[0, 1] + a1 * w_ref[1, 1], 0.0)
    nrm = jnp.sqrt(h0 * h0 + h1 * h1) + EPS
    o_ref[0:1, :] = h0 / nrm
    o_ref[1:2, :] = h1 / nrm


def _mid_tc(p0, p1, w):
    return pl.pallas_call(
        _mid_body,
        out_shape=jax.ShapeDtypeStruct((2, NPAD), jnp.float32),
        in_specs=[
            pl.BlockSpec((NC, NPAD), lambda: (0, 0)),
            pl.BlockSpec((NC, NPAD), lambda: (0, 0)),
            pl.BlockSpec(memory_space=pltpu.SMEM),
        ],
        out_specs=pl.BlockSpec((2, NPAD), lambda: (0, 0)),
    )(p0, p1, w)


def _post_body(p0_ref, p1_ref, w_ref, fw_ref, o_ref):
    a0 = _bf16(p0_ref[0:1, :] + p0_ref[1:2, :])
    a1 = _bf16(p1_ref[0:1, :] + p1_ref[1:2, :])
    h0 = jnp.maximum(a0 * w_ref[0, 0] + a1 * w_ref[1, 0], 0.0)
    h1 = jnp.maximum(a0 * w_ref[0, 1] + a1 * w_ref[1, 1], 0.0)
    o_ref[...] = jax.nn.sigmoid(_bf16(h0) * fw_ref[0] + _bf16(h1) * fw_ref[1])


def _post_tc(p0, p1, w, fw):
    return pl.pallas_call(
        _post_body,
        out_shape=jax.ShapeDtypeStruct((1, NPAD), jnp.float32),
        in_specs=[
            pl.BlockSpec((NC, NPAD), lambda: (0, 0)),
            pl.BlockSpec((NC, NPAD), lambda: (0, 0)),
            pl.BlockSpec(memory_space=pltpu.SMEM),
            pl.BlockSpec(memory_space=pltpu.SMEM),
        ],
        out_specs=pl.BlockSpec((1, NPAD), lambda: (0, 0)),
    )(p0, p1, w, fw)


def kernel(x, edge_index, W1, W2, final_weight):
    src = edge_index[0].astype(jnp.int32)
    dst = edge_index[1].astype(jnp.int32)
    xt = jnp.pad(x, ((0, NPAD - N_NODES), (0, 0))).T
    zeros = jnp.zeros((NPAD,), jnp.float32)

    xn = _normalize_tc(xt)
    p0, p1 = _propagate(xn[0], xn[1], zeros, src, dst)
    xn2 = _mid_tc(p0, p1, W1)
    q0, q1 = _propagate(xn2[0], xn2[1], zeros, src, dst)
    out = _post_tc(q0, q1, W2, final_weight)
    return out[0, :N_NODES]
